# Initial kernel scaffold; baseline (speedup 1.0000x reference)
import jax
import jax.numpy as jnp
from jax.experimental import pallas as pl


def kernel(x, edge_index, edge_attr, ptr, W1, b1, W2, b2):
    raise NotImplementedError("write your pallas kernel here")


# speedup vs baseline: 7.6831x; 7.6831x over previous
import functools

import jax
import jax.numpy as jnp
from jax import lax
from jax.experimental import pallas as pl
from jax.experimental.pallas import tpu as pltpu
from jax.experimental.pallas import tpu_sc as plsc

N = 10000
E = 320000
D = 128
DEPTH = 3

NC = 2
NS = 16
NW = NC * NS
EPT = E // NS
K = 80
CHUNKS = EPT // K
H = N // 2
HPAD = 5120
HPT = HPAD // NS
EPW = E // NW

_MESH = plsc.VectorSubcoreMesh(core_axis_name="c", subcore_axis_name="s")


def _make_sc_scatter(linear_src):
    scratch = [
        pltpu.VMEM((CHUNKS, K), jnp.int32),
        pltpu.VMEM((K, D), jnp.float32),
        pltpu.VMEM((K, D), jnp.float32),
        pltpu.VMEM((8, D), jnp.float32),
        pltpu.VMEM_SHARED((HPAD, D), jnp.float32),
        pltpu.SemaphoreType.DMA,
        pltpu.SemaphoreType.DMA,
    ]
    if not linear_src:
        scratch = [pltpu.VMEM((CHUNKS, K), jnp.int32)] + scratch

    @functools.partial(
        pl.kernel,
        mesh=_MESH,
        out_type=jax.ShapeDtypeStruct((NC, HPAD, D), jnp.float32),
        scratch_types=scratch,
    )
    def scat(*args):
        if linear_src:
            (table_hbm, didx_hbm, zeros_hbm, dep_hbm, out_hbm,
             didx_v, buf0, buf1, dep_v, acc, sem0, sem1) = args
            sidx_hbm = sidx_v = None
        else:
            (table_hbm, sidx_hbm, didx_hbm, zeros_hbm, dep_hbm, out_hbm,
             sidx_v, didx_v, buf0, buf1, dep_v, acc, sem0, sem1) = args
        cid = lax.axis_index("c")
        sid = lax.axis_index("s")

        pltpu.sync_copy(dep_hbm, dep_v)
        pltpu.sync_copy(zeros_hbm.at[pl.ds(sid * HPT, HPT)],
                        acc.at[pl.ds(sid * HPT, HPT)])

        if not linear_src:
            pltpu.sync_copy(sidx_hbm.at[sid], sidx_v)
        pltpu.sync_copy(didx_hbm.at[cid, sid], didx_v)
        plsc.subcore_barrier()

        def src_slice(j):
            if linear_src:
                start = pl.multiple_of(sid * EPT + j * K, 8)
                return table_hbm.at[pl.ds(start, K)]
            return table_hbm.at[sidx_v.at[j]]

        def gather(j, buf, sem):
            return pltpu.async_copy(src_slice(j), buf, sem)

        def wait(j, buf, sem):
            pltpu.make_async_copy(src_slice(j), buf, sem).wait()

        def scatter(j, buf):
            pltpu.sync_copy(buf, acc.at[didx_v.at[j]], add=True)

        gather(0, buf0, sem0)
        gather(1, buf1, sem1)

        def body(p, _):
            j0 = p * 2
            wait(j0, buf0, sem0)
            scatter(j0, buf0)
            gather(j0 + 2, buf0, sem0)
            wait(j0 + 1, buf1, sem1)
            scatter(j0 + 1, buf1)
            gather(j0 + 3, buf1, sem1)
            return _

        lax.fori_loop(0, CHUNKS // 2 - 1, body, None)
        jf = CHUNKS - 2
        wait(jf, buf0, sem0)
        scatter(jf, buf0)
        wait(jf + 1, buf1, sem1)
        scatter(jf + 1, buf1)

        plsc.subcore_barrier()
        pltpu.sync_copy(acc.at[pl.ds(sid * HPT, HPT)],
                        out_hbm.at[cid, pl.ds(sid * HPT, HPT)])

    return scat


DDEG = 128


@functools.partial(
    pl.kernel,
    mesh=_MESH,
    out_type=jax.ShapeDtypeStruct((NC, HPAD, DDEG), jnp.float32),
    scratch_types=[
        pltpu.VMEM((CHUNKS, K), jnp.int32),
        pltpu.VMEM((K, DDEG), jnp.float32),
        pltpu.VMEM_SHARED((HPAD, DDEG), jnp.float32),
    ],
)
def _sc_deg(didx_hbm, ones_hbm, zeros_hbm, out_hbm, didx_v, ones_v, acc):
    cid = lax.axis_index("c")
    sid = lax.axis_index("s")

    pltpu.sync_copy(zeros_hbm.at[pl.ds(sid * HPT, HPT)],
                    acc.at[pl.ds(sid * HPT, HPT)])
    pltpu.sync_copy(ones_hbm, ones_v)
    pltpu.sync_copy(didx_hbm.at[cid, sid], didx_v)
    plsc.subcore_barrier()

    def body(j, _):
        pltpu.sync_copy(ones_v, acc.at[didx_v.at[j]], add=True)
        return _

    lax.fori_loop(0, CHUNKS, body, None)

    plsc.subcore_barrier()
    pltpu.sync_copy(acc.at[pl.ds(sid * HPT, HPT)],
                    out_hbm.at[cid, pl.ds(sid * HPT, HPT)])


@functools.partial(
    pl.kernel,
    mesh=_MESH,
    out_type=jax.ShapeDtypeStruct((E,), jnp.float32),
    compiler_params=pltpu.CompilerParams(needs_layout_passes=False),
    scratch_types=[
        pltpu.VMEM((N,), jnp.float32),
        pltpu.VMEM((EPW,), jnp.int32),
        pltpu.VMEM((EPW,), jnp.float32),
    ],
)
def _sc_gather_dis(dis_hbm, row_hbm, out_hbm, dis_v, row_v, s_v):
    cid = lax.axis_index("c")
    sid = lax.axis_index("s")
    wid = cid * NS + sid

    pltpu.sync_copy(dis_hbm, dis_v)
    pltpu.sync_copy(row_hbm.at[pl.ds(wid * EPW, EPW)], row_v)

    def body(i, _):
        idx = row_v[pl.ds(i * 16, 16)]
        s_v[pl.ds(i * 16, 16)] = plsc.load_gather(dis_v, [idx])
        return _

    lax.fori_loop(0, EPW // 16, body, None)
    pltpu.sync_copy(s_v, out_hbm.at[pl.ds(wid * EPW, EPW)])


_sc_scat_e = _make_sc_scatter(linear_src=True)
_sc_scat_n = _make_sc_scatter(linear_src=False)


BN = 1000


def _tc_dis_body(deg_ref, x_ref, dis_ref, xp_ref):
    dis = jnp.sqrt(deg_ref[:, 0:1])
    dis_ref[...] = dis
    xp_ref[...] = x_ref[...] * dis


_tc_dis = pl.pallas_call(
    _tc_dis_body,
    grid=(N // BN,),
    in_specs=[
        pl.BlockSpec((BN, DDEG), lambda i: (i, 0)),
        pl.BlockSpec((BN, D), lambda i: (i, 0)),
    ],
    out_specs=[
        pl.BlockSpec((BN, 1), lambda i: (i, 0)),
        pl.BlockSpec((BN, D), lambda i: (i, 0)),
    ],
    out_shape=[
        jax.ShapeDtypeStruct((N, 1), jnp.float32),
        jax.ShapeDtypeStruct((N, D), jnp.float32),
    ],
)


BE = 4000


def _tc_scale_body(s_ref, ea_ref, out_ref):
    out_ref[...] = s_ref[...] * ea_ref[...]


_tc_scale = pl.pallas_call(
    _tc_scale_body,
    grid=(E // BE,),
    in_specs=[
        pl.BlockSpec((BE, 1), lambda i: (i, 0)),
        pl.BlockSpec((BE, D), lambda i: (i, 0)),
    ],
    out_specs=pl.BlockSpec((BE, D), lambda i: (i, 0)),
    out_shape=jax.ShapeDtypeStruct((E, D), jnp.float32),
)


def _tc_mlp_body(x_ref, ax_ref, ae_ref, dis_ref,
                 w1_ref, b1_ref, w2_ref, b2_ref, xn_ref, xpn_ref):
    dis = dis_ref[...]
    ax = ax_ref[...] * dis
    ae = ae_ref[...] * dis
    h = jnp.concatenate([x_ref[...], ax, ae], axis=1)
    h = jnp.dot(h, w1_ref[...], preferred_element_type=jnp.float32)
    h = h + b1_ref[...]
    h = 0.5 * h * (1.0 + lax.erf(h * 0.7071067811865476))
    xn = jnp.dot(h, w2_ref[...], preferred_element_type=jnp.float32)
    xn = xn + b2_ref[...]
    xn_ref[...] = xn
    xpn_ref[...] = xn * dis


_tc_mlp = pl.pallas_call(
    _tc_mlp_body,
    grid=(N // BN,),
    in_specs=[
        pl.BlockSpec((BN, D), lambda i: (i, 0)),
        pl.BlockSpec((BN, D), lambda i: (i, 0)),
        pl.BlockSpec((BN, D), lambda i: (i, 0)),
        pl.BlockSpec((BN, 1), lambda i: (i, 0)),
        pl.BlockSpec((3 * D, 2 * D), lambda i: (0, 0)),
        pl.BlockSpec((1, 2 * D), lambda i: (0, 0)),
        pl.BlockSpec((2 * D, D), lambda i: (0, 0)),
        pl.BlockSpec((1, D), lambda i: (0, 0)),
    ],
    out_specs=[
        pl.BlockSpec((BN, D), lambda i: (i, 0)),
        pl.BlockSpec((BN, D), lambda i: (i, 0)),
    ],
    out_shape=[
        jax.ShapeDtypeStruct((N, D), jnp.float32),
        jax.ShapeDtypeStruct((N, D), jnp.float32),
    ],
)


def _core_local_dst(col):
    eidx = jnp.arange(E, dtype=jnp.int32)
    trash = H + (eidx % (HPAD - H))
    locs = []
    for c in range(NC):
        local = col - c * H
        ok = (local >= 0) & (local < H)
        locs.append(jnp.where(ok, local, trash))
    return jnp.stack(locs).reshape(NC, NS, CHUNKS, K)


def kernel(x, edge_index, edge_attr, ptr, W1, b1, W2, b2):
    row = edge_index[0]
    col = edge_index[1]
    row3 = row.reshape(NS, CHUNKS, K)
    didx = _core_local_dst(col)
    zeros_d = jnp.zeros((HPAD, D), jnp.float32)
    zeros_g = jnp.zeros((HPAD, DDEG), jnp.float32)
    ones_g = jnp.ones((K, DDEG), jnp.float32)

    degp = _sc_deg(didx, ones_g, zeros_g)
    deg = degp[:, :H].reshape(N, DDEG)
    dis2, xp = _tc_dis(deg, x)
    dis1 = dis2.reshape(N)

    s = _sc_gather_dis(dis1, row)
    ep = _tc_scale(s.reshape(E, 1), edge_attr)
    pe = _sc_scat_e(ep, didx, zeros_d, jnp.zeros((8, D), jnp.float32))
    agge = pe[:, :H].reshape(N, D)

    px = None
    for layer in range(DEPTH):
        dep = pe[0, :8] if px is None else px[0, :8]
        px = _sc_scat_n(xp, row3, didx, zeros_d, dep)
        aggx = px[:, :H].reshape(N, D)
        x, xp = _tc_mlp(x, aggx, agge, dis2,
                        W1[layer], b1[layer].reshape(1, 2 * D),
                        W2[layer], b2[layer].reshape(1, D))
    return x


# speedup vs baseline: 8.3475x; 1.0865x over previous
import functools

import jax
import jax.numpy as jnp
from jax import lax
from jax.experimental import pallas as pl
from jax.experimental.pallas import tpu as pltpu
from jax.experimental.pallas import tpu_sc as plsc

N = 10000
E = 320000
D = 128
DEPTH = 3

NC = 2
NS = 16
NW = NC * NS
EPT = E // NS
K = 80
CHUNKS = EPT // K
H = N // 2
HPAD = 5120
HPT = HPAD // NS
EPW = E // NW

_MESH = plsc.VectorSubcoreMesh(core_axis_name="c", subcore_axis_name="s")


@functools.partial(
    pl.kernel,
    mesh=_MESH,
    out_type=jax.ShapeDtypeStruct((NC, NS, N), jnp.float32),
    compiler_params=pltpu.CompilerParams(needs_layout_passes=False),
    scratch_types=[
        pltpu.VMEM((N,), jnp.float32),
        pltpu.VMEM((EPW,), jnp.int32),
    ],
)
def _sc_deg(col_hbm, zeros_hbm, out_hbm, hist_v, col_v):
    cid = lax.axis_index("c")
    sid = lax.axis_index("s")

    pltpu.sync_copy(zeros_hbm, hist_v)
    pltpu.sync_copy(col_hbm.at[cid, sid], col_v)

    ones = jnp.ones((16,), jnp.float32)

    def body(i, _):
        idx = col_v[pl.ds(i * 16, 16)]
        plsc.addupdate_scatter(hist_v, [idx], ones)
        return _

    lax.fori_loop(0, EPW // 16, body, None)
    pltpu.sync_copy(hist_v, out_hbm.at[cid, sid])


@functools.partial(
    pl.kernel,
    mesh=_MESH,
    out_type=jax.ShapeDtypeStruct((E,), jnp.float32),
    compiler_params=pltpu.CompilerParams(needs_layout_passes=False),
    scratch_types=[
        pltpu.VMEM((N,), jnp.float32),
        pltpu.VMEM((EPW,), jnp.int32),
        pltpu.VMEM((EPW,), jnp.float32),
    ],
)
def _sc_gather_dis(dis_hbm, row_hbm, out_hbm, dis_v, row_v, s_v):
    cid = lax.axis_index("c")
    sid = lax.axis_index("s")
    wid = cid * NS + sid

    pltpu.sync_copy(dis_hbm, dis_v)
    pltpu.sync_copy(row_hbm.at[pl.ds(wid * EPW, EPW)], row_v)

    def body(i, _):
        idx = row_v[pl.ds(i * 16, 16)]
        s_v[pl.ds(i * 16, 16)] = plsc.load_gather(dis_v, [idx])
        return _

    lax.fori_loop(0, EPW // 16, body, None)
    pltpu.sync_copy(s_v, out_hbm.at[pl.ds(wid * EPW, EPW)])


def _make_sc_scatter(linear_src):
    scratch = [
        pltpu.VMEM((CHUNKS, K), jnp.int32),
        pltpu.VMEM((K, D), jnp.float32),
        pltpu.VMEM((K, D), jnp.float32),
        pltpu.VMEM((8, D), jnp.float32),
        pltpu.VMEM_SHARED((HPAD, D), jnp.float32),
        pltpu.SemaphoreType.DMA,
        pltpu.SemaphoreType.DMA,
    ]
    if not linear_src:
        scratch = [pltpu.VMEM((CHUNKS, K), jnp.int32)] + scratch

    @functools.partial(
        pl.kernel,
        mesh=_MESH,
        out_type=jax.ShapeDtypeStruct((NC, HPAD, D), jnp.float32),
        scratch_types=scratch,
    )
    def scat(*args):
        if linear_src:
            (table_hbm, didx_hbm, zeros_hbm, dep_hbm, out_hbm,
             didx_v, buf0, buf1, dep_v, acc, sem0, sem1) = args
            sidx_hbm = sidx_v = None
        else:
            (table_hbm, sidx_hbm, didx_hbm, zeros_hbm, dep_hbm, out_hbm,
             sidx_v, didx_v, buf0, buf1, dep_v, acc, sem0, sem1) = args
        cid = lax.axis_index("c")
        sid = lax.axis_index("s")

        pltpu.sync_copy(dep_hbm, dep_v)
        pltpu.sync_copy(zeros_hbm.at[pl.ds(sid * HPT, HPT)],
                        acc.at[pl.ds(sid * HPT, HPT)])

        if not linear_src:
            pltpu.sync_copy(sidx_hbm.at[sid], sidx_v)
        pltpu.sync_copy(didx_hbm.at[cid, sid], didx_v)
        plsc.subcore_barrier()

        def src_slice(j):
            if linear_src:
                start = pl.multiple_of(sid * EPT + j * K, 8)
                return table_hbm.at[pl.ds(start, K)]
            return table_hbm.at[sidx_v.at[j]]

        def gather(j, buf, sem):
            return pltpu.async_copy(src_slice(j), buf, sem)

        def wait(j, buf, sem):
            pltpu.make_async_copy(src_slice(j), buf, sem).wait()

        def scatter(j, buf):
            pltpu.sync_copy(buf, acc.at[didx_v.at[j]], add=True)

        gather(0, buf0, sem0)
        gather(1, buf1, sem1)

        def body(p, _):
            j0 = p * 2
            wait(j0, buf0, sem0)
            scatter(j0, buf0)
            gather(j0 + 2, buf0, sem0)
            wait(j0 + 1, buf1, sem1)
            scatter(j0 + 1, buf1)
            gather(j0 + 3, buf1, sem1)
            return _

        lax.fori_loop(0, CHUNKS // 2 - 1, body, None)
        jf = CHUNKS - 2
        wait(jf, buf0, sem0)
        scatter(jf, buf0)
        wait(jf + 1, buf1, sem1)
        scatter(jf + 1, buf1)

        plsc.subcore_barrier()
        pltpu.sync_copy(acc.at[pl.ds(sid * HPT, HPT)],
                        out_hbm.at[cid, pl.ds(sid * HPT, HPT)])

    return scat


_sc_scat_e = _make_sc_scatter(linear_src=True)
_sc_scat_n = _make_sc_scatter(linear_src=False)


BN = 1000


def _tc_dis_body(deg_ref, x_ref, dis_ref, xp_ref):
    dis = jnp.sqrt(deg_ref[...])
    dis_ref[...] = dis
    xp_ref[...] = x_ref[...] * dis


_tc_dis = pl.pallas_call(
    _tc_dis_body,
    grid=(N // BN,),
    in_specs=[
        pl.BlockSpec((BN, 1), lambda i: (i, 0)),
        pl.BlockSpec((BN, D), lambda i: (i, 0)),
    ],
    out_specs=[
        pl.BlockSpec((BN, 1), lambda i: (i, 0)),
        pl.BlockSpec((BN, D), lambda i: (i, 0)),
    ],
    out_shape=[
        jax.ShapeDtypeStruct((N, 1), jnp.float32),
        jax.ShapeDtypeStruct((N, D), jnp.float32),
    ],
)


BE = 4000


def _tc_scale_body(s_ref, ea_ref, out_ref):
    out_ref[...] = s_ref[...] * ea_ref[...]


_tc_scale = pl.pallas_call(
    _tc_scale_body,
    grid=(E // BE,),
    in_specs=[
        pl.BlockSpec((BE, 1), lambda i: (i, 0)),
        pl.BlockSpec((BE, D), lambda i: (i, 0)),
    ],
    out_specs=pl.BlockSpec((BE, D), lambda i: (i, 0)),
    out_shape=jax.ShapeDtypeStruct((E, D), jnp.float32),
)


def _tc_mlp_body(x_ref, ax_ref, ae_ref, dis_ref,
                 w1_ref, b1_ref, w2_ref, b2_ref, xn_ref, xpn_ref):
    dis = dis_ref[...]
    ax = ax_ref[...] * dis
    ae = ae_ref[...] * dis
    h = jnp.concatenate([x_ref[...], ax, ae], axis=1)
    h = jnp.dot(h, w1_ref[...], preferred_element_type=jnp.float32)
    h = h + b1_ref[...]
    h = 0.5 * h * (1.0 + lax.erf(h * 0.7071067811865476))
    xn = jnp.dot(h, w2_ref[...], preferred_element_type=jnp.float32)
    xn = xn + b2_ref[...]
    xn_ref[...] = xn
    xpn_ref[...] = xn * dis


_tc_mlp = pl.pallas_call(
    _tc_mlp_body,
    grid=(N // BN,),
    in_specs=[
        pl.BlockSpec((BN, D), lambda i: (i, 0)),
        pl.BlockSpec((BN, D), lambda i: (i, 0)),
        pl.BlockSpec((BN, D), lambda i: (i, 0)),
        pl.BlockSpec((BN, 1), lambda i: (i, 0)),
        pl.BlockSpec((3 * D, 2 * D), lambda i: (0, 0)),
        pl.BlockSpec((1, 2 * D), lambda i: (0, 0)),
        pl.BlockSpec((2 * D, D), lambda i: (0, 0)),
        pl.BlockSpec((1, D), lambda i: (0, 0)),
    ],
    out_specs=[
        pl.BlockSpec((BN, D), lambda i: (i, 0)),
        pl.BlockSpec((BN, D), lambda i: (i, 0)),
    ],
    out_shape=[
        jax.ShapeDtypeStruct((N, D), jnp.float32),
        jax.ShapeDtypeStruct((N, D), jnp.float32),
    ],
)


def _core_local_dst(col):
    eidx = jnp.arange(E, dtype=jnp.int32)
    trash = H + (eidx % (HPAD - H))
    locs = []
    for c in range(NC):
        local = col - c * H
        ok = (local >= 0) & (local < H)
        locs.append(jnp.where(ok, local, trash))
    return jnp.stack(locs).reshape(NC, NS, CHUNKS, K)


def kernel(x, edge_index, edge_attr, ptr, W1, b1, W2, b2):
    row = edge_index[0]
    col = edge_index[1]
    row3 = row.reshape(NS, CHUNKS, K)
    colw = col.reshape(NC, NS, EPW)
    didx = _core_local_dst(col)
    zeros_d = jnp.zeros((HPAD, D), jnp.float32)
    zeros_n = jnp.zeros((N,), jnp.float32)

    hists = _sc_deg(colw, zeros_n)
    deg = hists.sum(axis=(0, 1)).reshape(N, 1)
    dis2, xp = _tc_dis(deg, x)
    dis1 = dis2.reshape(N)

    s = _sc_gather_dis(dis1, row)
    ep = _tc_scale(s.reshape(E, 1), edge_attr)
    pe = _sc_scat_e(ep, didx, zeros_d, jnp.zeros((8, D), jnp.float32))
    agge = pe[:, :H].reshape(N, D)

    px = None
    for layer in range(DEPTH):
        dep = pe[0, :8] if px is None else px[0, :8]
        px = _sc_scat_n(xp, row3, didx, zeros_d, dep)
        aggx = px[:, :H].reshape(N, D)
        x, xp = _tc_mlp(x, aggx, agge, dis2,
                        W1[layer], b1[layer].reshape(1, 2 * D),
                        W2[layer], b2[layer].reshape(1, D))
    return x
